# Initial kernel scaffold; baseline (speedup 1.0000x reference)
import functools

import jax
import jax.numpy as jnp
from jax import lax
from jax.experimental import pallas as pl
from jax.experimental.pallas import tpu as pltpu
from jax.experimental.pallas import tpu_sc as plsc

N_NODES = 10000
NP = 10240
D_FEAT = 256
D_OUT = 16
WIN = 128
NC = 2
NS = 16
ROWS_PER_SUB = NP // NS

_mesh = plsc.VectorSubcoreMesh(core_axis_name="c", subcore_axis_name="s")


def _matmul_body(x_ref, w_ref, h_ref):
    h_ref[:, :] = lax.dot_general(
        x_ref[:, :], w_ref[:, :], (((1,), (0,)), ((), ())),
        precision=lax.Precision.HIGHEST, preferred_element_type=jnp.float32)


def _matmul(x_p, W):
    blk = 1280
    return pl.pallas_call(
        _matmul_body,
        grid=(NP // blk,),
        in_specs=[
            pl.BlockSpec((blk, D_FEAT), lambda i: (i, 0)),
            pl.BlockSpec((D_FEAT, D_OUT), lambda i: (0, 0)),
        ],
        out_specs=pl.BlockSpec((blk, D_OUT), lambda i: (i, 0)),
        out_shape=jax.ShapeDtypeStruct((NP, D_OUT), jnp.float32),
    )(x_p, W)


def _deg_body(nwin, dst_hbm, degp_hbm, deg_sh, idx_v, ones_v, zero_v):
    cid = lax.axis_index("c")
    sid = lax.axis_index("s")
    wid = cid * NS + sid

    @pl.loop(0, ROWS_PER_SUB, step=16)
    def _(i):
        zero_v[pl.ds(i, 16)] = jnp.zeros((16,), jnp.float32)

    @pl.loop(0, WIN, step=16)
    def _(i):
        ones_v[pl.ds(i, 16)] = jnp.ones((16,), jnp.float32)

    pltpu.sync_copy(zero_v, deg_sh.at[pl.ds(sid * ROWS_PER_SUB, ROWS_PER_SUB)])
    plsc.subcore_barrier()

    @pl.loop(wid, nwin, step=NC * NS)
    def _(w):
        pltpu.sync_copy(dst_hbm.at[pl.ds(w * WIN, WIN)], idx_v.at[0])
        pltpu.sync_copy(ones_v, deg_sh.at[idx_v.at[0]], add=True)

    plsc.subcore_barrier()
    sl = pl.ds(sid * ROWS_PER_SUB, ROWS_PER_SUB)
    pltpu.sync_copy(deg_sh.at[sl], degp_hbm.at[cid, sl])


def _degrees(dst):
    nwin = dst.shape[0] // WIN
    kern = pl.kernel(
        functools.partial(_deg_body, nwin),
        out_type=jax.ShapeDtypeStruct((NC, NP), jnp.float32),
        mesh=_mesh,
        scratch_types=[
            pltpu.VMEM_SHARED((NP,), jnp.float32),
            pltpu.VMEM((1, WIN), jnp.int32),
            pltpu.VMEM((WIN,), jnp.float32),
            pltpu.VMEM((ROWS_PER_SUB,), jnp.float32),
        ],
    )
    return kern(dst)


def _scale_body(degt_ref, h_ref, g_ref, dinv_ref):
    deg = degt_ref[:, 0:1] + degt_ref[:, 1:2] + 1.0
    dinv = lax.rsqrt(deg)
    dinv_ref[:, :] = dinv
    g_ref[:, :] = h_ref[:, :] * dinv


def _scale(degt, h):
    return pl.pallas_call(
        _scale_body,
        in_specs=[
            pl.BlockSpec((NP, 2), lambda: (0, 0)),
            pl.BlockSpec((NP, D_OUT), lambda: (0, 0)),
        ],
        out_specs=[
            pl.BlockSpec((NP, D_OUT), lambda: (0, 0)),
            pl.BlockSpec((NP, 1), lambda: (0, 0)),
        ],
        out_shape=[
            jax.ShapeDtypeStruct((NP, D_OUT), jnp.float32),
            jax.ShapeDtypeStruct((NP, 1), jnp.float32),
        ],
    )(degt, h)


def _agg_body(nwin, src_hbm, dst_hbm, g_hbm, accp_hbm,
              acc_sh, sidx_v, didx_v, rows_v, zb_v):
    cid = lax.axis_index("c")
    sid = lax.axis_index("s")
    wid = cid * NS + sid

    @pl.loop(0, WIN)
    def _(i):
        zb_v[i, :] = jnp.zeros((16,), jnp.float32)

    for k in range(ROWS_PER_SUB // WIN):
        pltpu.sync_copy(zb_v, acc_sh.at[pl.ds(sid * ROWS_PER_SUB + k * WIN, WIN)])
    plsc.subcore_barrier()

    @pl.loop(wid, nwin, step=NC * NS)
    def _(w):
        base = w * WIN
        pltpu.sync_copy(src_hbm.at[pl.ds(base, WIN)], sidx_v.at[0])
        pltpu.sync_copy(dst_hbm.at[pl.ds(base, WIN)], didx_v.at[0])
        pltpu.sync_copy(g_hbm.at[sidx_v.at[0]], rows_v)
        pltpu.sync_copy(rows_v, acc_sh.at[didx_v.at[0]], add=True)

    plsc.subcore_barrier()
    sl = pl.ds(sid * ROWS_PER_SUB, ROWS_PER_SUB)
    pltpu.sync_copy(acc_sh.at[sl], accp_hbm.at[cid, sl])


def _aggregate(src, dst, g):
    nwin = src.shape[0] // WIN
    kern = pl.kernel(
        functools.partial(_agg_body, nwin),
        out_type=jax.ShapeDtypeStruct((NC, NP, D_OUT), jnp.float32),
        mesh=_mesh,
        scratch_types=[
            pltpu.VMEM_SHARED((NP, D_OUT), jnp.float32),
            pltpu.VMEM((1, WIN), jnp.int32),
            pltpu.VMEM((1, WIN), jnp.int32),
            pltpu.VMEM((WIN, D_OUT), jnp.float32),
            pltpu.VMEM((WIN, D_OUT), jnp.float32),
        ],
    )
    return kern(src, dst, g)


def _epilogue_body(dinv_ref, g_ref, a0_ref, a1_ref, b_ref, out_ref):
    acc = g_ref[:, :] + a0_ref[:, :] + a1_ref[:, :]
    out_ref[:, :] = dinv_ref[:, :] * acc + b_ref[:, :]


def _epilogue(dinv, g, accp, b2):
    full = lambda: (0, 0)
    return pl.pallas_call(
        _epilogue_body,
        in_specs=[
            pl.BlockSpec((NP, 1), full),
            pl.BlockSpec((NP, D_OUT), full),
            pl.BlockSpec((NP, D_OUT), full),
            pl.BlockSpec((NP, D_OUT), full),
            pl.BlockSpec((1, D_OUT), full),
        ],
        out_specs=pl.BlockSpec((NP, D_OUT), full),
        out_shape=jax.ShapeDtypeStruct((NP, D_OUT), jnp.float32),
    )(dinv, g, accp[0], accp[1], b2)


def kernel(x, edge_index, W, b):
    n, e = x.shape[0], edge_index.shape[1]
    src = edge_index[0].astype(jnp.int32)
    dst = edge_index[1].astype(jnp.int32)
    if e % WIN:
        pad = WIN - e % WIN
        src = jnp.concatenate([src, jnp.full((pad,), NP - 1, jnp.int32)])
        dst = jnp.concatenate([dst, jnp.full((pad,), NP - 1, jnp.int32)])

    x_p = jnp.pad(x, ((0, NP - n), (0, 0)))
    h = _matmul(x_p, W)
    degp = _degrees(dst)
    degt = jnp.transpose(degp)
    g, dinv = _scale(degt, h)
    accp = _aggregate(src, dst, g)
    out_p = _epilogue(dinv, g, accp, jnp.reshape(b, (1, D_OUT)))
    return out_p[:n]


# speedup vs baseline: 20.6920x; 20.6920x over previous
import functools

import jax
import jax.numpy as jnp
from jax import lax
from jax.experimental import pallas as pl
from jax.experimental.pallas import tpu as pltpu
from jax.experimental.pallas import tpu_sc as plsc

N_NODES = 10000
NP = 10240
D_FEAT = 256
D_OUT = 16
WIN = 128
NC = 2
NS = 16
ROWS_PER_SUB = NP // NS

_mesh = plsc.VectorSubcoreMesh(core_axis_name="c", subcore_axis_name="s")
_sc_params = pltpu.CompilerParams(use_tc_tiling_on_sc=False)


def _matmul_body(x_ref, w_ref, h_ref):
    h_ref[:, :] = lax.dot_general(
        x_ref[:, :], w_ref[:, :], (((1,), (0,)), ((), ())),
        precision=lax.Precision.HIGHEST, preferred_element_type=jnp.float32)


def _matmul(x_p, W):
    blk = 1280
    return pl.pallas_call(
        _matmul_body,
        grid=(NP // blk,),
        in_specs=[
            pl.BlockSpec((blk, D_FEAT), lambda i: (i, 0)),
            pl.BlockSpec((D_FEAT, D_OUT), lambda i: (0, 0)),
        ],
        out_specs=pl.BlockSpec((blk, D_OUT), lambda i: (i, 0)),
        out_shape=jax.ShapeDtypeStruct((NP, D_OUT), jnp.float32),
    )(x_p, W)


def _deg_body(nwin, dst_hbm, degp_hbm, deg_sh, idx_v, ones_v, zero_v):
    cid = lax.axis_index("c")
    sid = lax.axis_index("s")
    wid = cid * NS + sid

    @pl.loop(0, ROWS_PER_SUB, step=16)
    def _(i):
        zero_v[pl.ds(i, 16)] = jnp.zeros((16,), jnp.float32)

    @pl.loop(0, WIN, step=16)
    def _(i):
        ones_v[pl.ds(i, 16)] = jnp.ones((16,), jnp.float32)

    pltpu.sync_copy(zero_v, deg_sh.at[pl.ds(sid * ROWS_PER_SUB, ROWS_PER_SUB)])
    plsc.subcore_barrier()

    @pl.loop(wid, nwin, step=NC * NS)
    def _(w):
        pltpu.sync_copy(dst_hbm.at[pl.ds(w * WIN, WIN)], idx_v.at[0])
        pltpu.sync_copy(ones_v, deg_sh.at[idx_v.at[0]], add=True)

    plsc.subcore_barrier()
    sl = pl.ds(sid * ROWS_PER_SUB, ROWS_PER_SUB)
    pltpu.sync_copy(deg_sh.at[sl], degp_hbm.at[cid, sl])


def _degrees(dst):
    nwin = dst.shape[0] // WIN
    kern = pl.kernel(
        functools.partial(_deg_body, nwin),
        out_type=jax.ShapeDtypeStruct((NC, NP), jnp.float32),
        mesh=_mesh,
        scratch_types=[
            pltpu.VMEM_SHARED((NP,), jnp.float32),
            pltpu.VMEM((1, WIN), jnp.int32),
            pltpu.VMEM((WIN,), jnp.float32),
            pltpu.VMEM((ROWS_PER_SUB,), jnp.float32),
        ],
        compiler_params=_sc_params,
    )
    return kern(dst)


def _scale_body(degt_ref, h_ref, g_ref, dinv_ref):
    deg = degt_ref[:, 0:1] + degt_ref[:, 1:2] + 1.0
    dinv = lax.rsqrt(deg)
    dinv_ref[:, :] = dinv
    g_ref[:, :] = h_ref[:, :] * dinv


def _scale(degt, h):
    return pl.pallas_call(
        _scale_body,
        in_specs=[
            pl.BlockSpec((NP, 2), lambda: (0, 0)),
            pl.BlockSpec((NP, D_OUT), lambda: (0, 0)),
        ],
        out_specs=[
            pl.BlockSpec((NP, D_OUT), lambda: (0, 0)),
            pl.BlockSpec((NP, 1), lambda: (0, 0)),
        ],
        out_shape=[
            jax.ShapeDtypeStruct((NP, D_OUT), jnp.float32),
            jax.ShapeDtypeStruct((NP, 1), jnp.float32),
        ],
    )(degt, h)


def _agg_body(nwin, src_hbm, dst_hbm, g_hbm, accp_hbm,
              acc_sh, sidx_v, didx_v, rows_v, zb_v):
    cid = lax.axis_index("c")
    sid = lax.axis_index("s")
    wid = cid * NS + sid

    @pl.loop(0, WIN)
    def _(i):
        zb_v[i, :] = jnp.zeros((16,), jnp.float32)

    for k in range(ROWS_PER_SUB // WIN):
        pltpu.sync_copy(zb_v, acc_sh.at[pl.ds(sid * ROWS_PER_SUB + k * WIN, WIN)])
    plsc.subcore_barrier()

    @pl.loop(wid, nwin, step=NC * NS)
    def _(w):
        base = w * WIN
        pltpu.sync_copy(src_hbm.at[pl.ds(base, WIN)], sidx_v.at[0])
        pltpu.sync_copy(dst_hbm.at[pl.ds(base, WIN)], didx_v.at[0])
        pltpu.sync_copy(g_hbm.at[sidx_v.at[0]], rows_v)
        pltpu.sync_copy(rows_v, acc_sh.at[didx_v.at[0]], add=True)

    plsc.subcore_barrier()
    sl = pl.ds(sid * ROWS_PER_SUB, ROWS_PER_SUB)
    pltpu.sync_copy(acc_sh.at[sl], accp_hbm.at[cid, sl])


def _aggregate(src, dst, g):
    nwin = src.shape[0] // WIN
    kern = pl.kernel(
        functools.partial(_agg_body, nwin),
        out_type=jax.ShapeDtypeStruct((NC, NP, D_OUT), jnp.float32),
        mesh=_mesh,
        scratch_types=[
            pltpu.VMEM_SHARED((NP, D_OUT), jnp.float32),
            pltpu.VMEM((1, WIN), jnp.int32),
            pltpu.VMEM((1, WIN), jnp.int32),
            pltpu.VMEM((WIN, D_OUT), jnp.float32),
            pltpu.VMEM((WIN, D_OUT), jnp.float32),
        ],
        compiler_params=_sc_params,
    )
    return kern(src, dst, g)


def _epilogue_body(dinv_ref, g_ref, a0_ref, a1_ref, b_ref, out_ref):
    acc = g_ref[:, :] + a0_ref[:, :] + a1_ref[:, :]
    out_ref[:, :] = dinv_ref[:, :] * acc + b_ref[:, :]


def _epilogue(dinv, g, accp, b2):
    full = lambda: (0, 0)
    return pl.pallas_call(
        _epilogue_body,
        in_specs=[
            pl.BlockSpec((NP, 1), full),
            pl.BlockSpec((NP, D_OUT), full),
            pl.BlockSpec((NP, D_OUT), full),
            pl.BlockSpec((NP, D_OUT), full),
            pl.BlockSpec((1, D_OUT), full),
        ],
        out_specs=pl.BlockSpec((NP, D_OUT), full),
        out_shape=jax.ShapeDtypeStruct((NP, D_OUT), jnp.float32),
    )(dinv, g, accp[0], accp[1], b2)


def kernel(x, edge_index, W, b):
    n, e = x.shape[0], edge_index.shape[1]
    src = edge_index[0].astype(jnp.int32)
    dst = edge_index[1].astype(jnp.int32)
    if e % WIN:
        pad = WIN - e % WIN
        src = jnp.concatenate([src, jnp.full((pad,), NP - 1, jnp.int32)])
        dst = jnp.concatenate([dst, jnp.full((pad,), NP - 1, jnp.int32)])

    x_p = jnp.pad(x, ((0, NP - n), (0, 0)))
    h = _matmul(x_p, W)
    degp = _degrees(dst)
    degt = jnp.transpose(degp)
    g, dinv = _scale(degt, h)
    accp = _aggregate(src, dst, g)
    out_p = _epilogue(dinv, g, accp, jnp.reshape(b, (1, D_OUT)))
    return out_p[:n]


# speedup vs baseline: 25.5441x; 1.2345x over previous
import functools

import jax
import jax.numpy as jnp
from jax import lax
from jax.experimental import pallas as pl
from jax.experimental.pallas import tpu as pltpu
from jax.experimental.pallas import tpu_sc as plsc

NP = 10240
D_FEAT = 256
D_OUT = 16
WIN = 128
NC = 2
NS = 16
NW = NC * NS
WPS = 40
E_TILE = NW * WPS * WIN
ROWS_PER_SUB = NP // NS

_mesh = plsc.VectorSubcoreMesh(core_axis_name="c", subcore_axis_name="s")
_sc_params = pltpu.CompilerParams(use_tc_tiling_on_sc=False)


def _matmul_body(x_ref, w_ref, h_ref):
    h_ref[:, :] = lax.dot_general(
        x_ref[:, :], w_ref[:, :], (((1,), (0,)), ((), ())),
        precision=lax.Precision.HIGHEST, preferred_element_type=jnp.float32)


def _matmul(x_p, W):
    blk = 1280
    return pl.pallas_call(
        _matmul_body,
        grid=(NP // blk,),
        in_specs=[
            pl.BlockSpec((blk, D_FEAT), lambda i: (i, 0)),
            pl.BlockSpec((D_FEAT, D_OUT), lambda i: (0, 0)),
        ],
        out_specs=pl.BlockSpec((blk, D_OUT), lambda i: (i, 0)),
        out_shape=jax.ShapeDtypeStruct((NP, D_OUT), jnp.float32),
    )(x_p, W)


def _deg_body(nwps, dst_hbm, degp_hbm, deg_sh, idx_v, ones_v, zero_v,
              sem_i, sem_w):
    cid = lax.axis_index("c")
    sid = lax.axis_index("s")
    wid = cid * NS + sid

    fetch = pltpu.async_copy(dst_hbm.at[pl.ds(wid * nwps, nwps), :], idx_v,
                             sem_i)

    @pl.loop(0, ROWS_PER_SUB, step=16)
    def _(i):
        zero_v[pl.ds(i, 16)] = jnp.zeros((16,), jnp.float32)

    @pl.loop(0, WIN, step=16)
    def _(i):
        ones_v[pl.ds(i, 16)] = jnp.ones((16,), jnp.float32)

    pltpu.sync_copy(zero_v, deg_sh.at[pl.ds(sid * ROWS_PER_SUB, ROWS_PER_SUB)])
    fetch.wait()
    plsc.subcore_barrier()

    for j in range(nwps):
        pltpu.async_copy(ones_v, deg_sh.at[idx_v.at[j]], sem_w,
                         add=True).wait()

    plsc.subcore_barrier()
    sl = pl.ds(sid * ROWS_PER_SUB, ROWS_PER_SUB)
    pltpu.sync_copy(deg_sh.at[sl], degp_hbm.at[cid, sl])


def _degrees(dst2d):
    nwps = dst2d.shape[0] // NW
    kern = pl.kernel(
        functools.partial(_deg_body, nwps),
        out_type=jax.ShapeDtypeStruct((NC, NP), jnp.float32),
        mesh=_mesh,
        scratch_types=[
            pltpu.VMEM_SHARED((NP,), jnp.float32),
            pltpu.VMEM((nwps, WIN), jnp.int32),
            pltpu.VMEM((WIN,), jnp.float32),
            pltpu.VMEM((ROWS_PER_SUB,), jnp.float32),
            pltpu.SemaphoreType.DMA,
            pltpu.SemaphoreType.DMA,
        ],
        compiler_params=_sc_params,
    )
    return kern(dst2d)


def _scale_body(degt_ref, h_ref, g_ref, dinv_ref):
    deg = degt_ref[:, 0:1] + degt_ref[:, 1:2] + 1.0
    dinv = lax.rsqrt(deg)
    dinv_ref[:, :] = dinv
    g_ref[:, :] = h_ref[:, :] * dinv


def _scale(degt, h):
    return pl.pallas_call(
        _scale_body,
        in_specs=[
            pl.BlockSpec((NP, 2), lambda: (0, 0)),
            pl.BlockSpec((NP, D_OUT), lambda: (0, 0)),
        ],
        out_specs=[
            pl.BlockSpec((NP, D_OUT), lambda: (0, 0)),
            pl.BlockSpec((NP, 1), lambda: (0, 0)),
        ],
        out_shape=[
            jax.ShapeDtypeStruct((NP, D_OUT), jnp.float32),
            jax.ShapeDtypeStruct((NP, 1), jnp.float32),
        ],
    )(degt, h)


def _agg_body(nwps, src_hbm, dst_hbm, g_hbm, accp_hbm,
              acc_sh, sidx_v, didx_v, buf0, buf1, zb_v,
              sem_i, sem_j, sem_g0, sem_g1, sem_s0, sem_s1):
    cid = lax.axis_index("c")
    sid = lax.axis_index("s")
    wid = cid * NS + sid

    fs = pltpu.async_copy(src_hbm.at[pl.ds(wid * nwps, nwps), :], sidx_v, sem_i)
    fd = pltpu.async_copy(dst_hbm.at[pl.ds(wid * nwps, nwps), :], didx_v, sem_j)

    @pl.loop(0, WIN)
    def _(i):
        zb_v[i, :] = jnp.zeros((16,), jnp.float32)

    for k in range(ROWS_PER_SUB // WIN):
        pltpu.sync_copy(zb_v, acc_sh.at[pl.ds(sid * ROWS_PER_SUB + k * WIN, WIN)])
    fs.wait()
    fd.wait()
    plsc.subcore_barrier()

    for j in range(nwps):
        pltpu.async_copy(g_hbm.at[sidx_v.at[j]], buf0, sem_g0).wait()
        pltpu.async_copy(buf0, acc_sh.at[didx_v.at[j]], sem_s0,
                         add=True).wait()

    plsc.subcore_barrier()
    sl = pl.ds(sid * ROWS_PER_SUB, ROWS_PER_SUB)
    pltpu.sync_copy(acc_sh.at[sl], accp_hbm.at[cid, sl])


def _aggregate(src2d, dst2d, g):
    nwps = src2d.shape[0] // NW
    kern = pl.kernel(
        functools.partial(_agg_body, nwps),
        out_type=jax.ShapeDtypeStruct((NC, NP, D_OUT), jnp.float32),
        mesh=_mesh,
        scratch_types=[
            pltpu.VMEM_SHARED((NP, D_OUT), jnp.float32),
            pltpu.VMEM((nwps, WIN), jnp.int32),
            pltpu.VMEM((nwps, WIN), jnp.int32),
            pltpu.VMEM((WIN, D_OUT), jnp.float32),
            pltpu.VMEM((WIN, D_OUT), jnp.float32),
            pltpu.VMEM((WIN, D_OUT), jnp.float32),
        ] + [pltpu.SemaphoreType.DMA] * 6,
        compiler_params=_sc_params,
    )
    return kern(src2d, dst2d, g)


def _epilogue_body(dinv_ref, g_ref, a0_ref, a1_ref, b_ref, out_ref):
    acc = g_ref[:, :] + a0_ref[:, :] + a1_ref[:, :]
    out_ref[:, :] = dinv_ref[:, :] * acc + b_ref[:, :]


def _epilogue(dinv, g, accp, b2):
    full = lambda: (0, 0)
    return pl.pallas_call(
        _epilogue_body,
        in_specs=[
            pl.BlockSpec((NP, 1), full),
            pl.BlockSpec((NP, D_OUT), full),
            pl.BlockSpec((NP, D_OUT), full),
            pl.BlockSpec((NP, D_OUT), full),
            pl.BlockSpec((1, D_OUT), full),
        ],
        out_specs=pl.BlockSpec((NP, D_OUT), full),
        out_shape=jax.ShapeDtypeStruct((NP, D_OUT), jnp.float32),
    )(dinv, g, accp[0], accp[1], b2)


def kernel(x, edge_index, W, b):
    n, e = x.shape[0], edge_index.shape[1]
    src = edge_index[0].astype(jnp.int32)
    dst = edge_index[1].astype(jnp.int32)
    ep = -(-e // E_TILE) * E_TILE
    if ep != e:
        sent = jnp.full((ep - e,), NP - 1, jnp.int32)
        src = jnp.concatenate([src, sent])
        dst = jnp.concatenate([dst, sent])
    src2d = src.reshape(ep // WIN, WIN)
    dst2d = dst.reshape(ep // WIN, WIN)

    x_p = jnp.pad(x, ((0, NP - n), (0, 0)))
    h = _matmul(x_p, W)
    degp = _degrees(dst2d)
    degt = jnp.transpose(degp)
    g, dinv = _scale(degt, h)
    accp = _aggregate(src2d, dst2d, g)
    out_p = _epilogue(dinv, g, accp, jnp.reshape(b, (1, D_OUT)))
    return out_p[:n]


# speedup vs baseline: 28.0896x; 1.0996x over previous
import functools

import jax
import jax.numpy as jnp
from jax import lax
from jax.experimental import pallas as pl
from jax.experimental.pallas import tpu as pltpu
from jax.experimental.pallas import tpu_sc as plsc

NP = 10240
D_FEAT = 256
D_OUT = 16
WIN = 128
NC = 2
NS = 16
NW = NC * NS
WPS = 40
E_TILE = NW * WPS * WIN
ROWS_PER_SUB = NP // NS

_mesh = plsc.VectorSubcoreMesh(core_axis_name="c", subcore_axis_name="s")
_sc_params = pltpu.CompilerParams(use_tc_tiling_on_sc=False)


def _matmul_body(x_ref, w_ref, h_ref):
    h_ref[:, :] = lax.dot_general(
        x_ref[:, :], w_ref[:, :], (((1,), (0,)), ((), ())),
        precision=lax.Precision.HIGHEST, preferred_element_type=jnp.float32)


def _matmul(x_p, W):
    blk = 1280
    return pl.pallas_call(
        _matmul_body,
        grid=(NP // blk,),
        in_specs=[
            pl.BlockSpec((blk, D_FEAT), lambda i: (i, 0)),
            pl.BlockSpec((D_FEAT, D_OUT), lambda i: (0, 0)),
        ],
        out_specs=pl.BlockSpec((blk, D_OUT), lambda i: (i, 0)),
        out_shape=jax.ShapeDtypeStruct((NP, D_OUT), jnp.float32),
    )(x_p, W)


def _deg_body(nwps, dst_hbm, degp_hbm, deg_sh, idx_v, ones_v, zero_v,
              sem_i, sem_w):
    cid = lax.axis_index("c")
    sid = lax.axis_index("s")
    wid = cid * NS + sid

    fetch = pltpu.async_copy(dst_hbm.at[pl.ds(wid * nwps, nwps), :], idx_v,
                             sem_i)

    @pl.loop(0, ROWS_PER_SUB, step=16)
    def _(i):
        zero_v[pl.ds(i, 16)] = jnp.zeros((16,), jnp.float32)

    @pl.loop(0, WIN, step=16)
    def _(i):
        ones_v[pl.ds(i, 16)] = jnp.ones((16,), jnp.float32)

    pltpu.sync_copy(zero_v, deg_sh.at[pl.ds(sid * ROWS_PER_SUB, ROWS_PER_SUB)])
    fetch.wait()
    plsc.subcore_barrier()

    for j in range(nwps):
        pltpu.async_copy(ones_v, deg_sh.at[idx_v.at[j]], sem_w,
                         add=True).wait()

    plsc.subcore_barrier()
    sl = pl.ds(sid * ROWS_PER_SUB, ROWS_PER_SUB)
    pltpu.sync_copy(deg_sh.at[sl], degp_hbm.at[cid, sl])


def _degrees(dst2d):
    nwps = dst2d.shape[0] // NW
    kern = pl.kernel(
        functools.partial(_deg_body, nwps),
        out_type=jax.ShapeDtypeStruct((NC, NP), jnp.float32),
        mesh=_mesh,
        scratch_types=[
            pltpu.VMEM_SHARED((NP,), jnp.float32),
            pltpu.VMEM((nwps, WIN), jnp.int32),
            pltpu.VMEM((WIN,), jnp.float32),
            pltpu.VMEM((ROWS_PER_SUB,), jnp.float32),
            pltpu.SemaphoreType.DMA,
            pltpu.SemaphoreType.DMA,
        ],
        compiler_params=_sc_params,
    )
    return kern(dst2d)


def _scale_body(degt_ref, h_ref, g_ref, dinv_ref):
    deg = degt_ref[:, 0:1] + degt_ref[:, 1:2] + 1.0
    dinv = lax.rsqrt(deg)
    dinv_ref[:, :] = dinv
    g_ref[:, :] = h_ref[:, :] * dinv


def _scale(degt, h):
    return pl.pallas_call(
        _scale_body,
        in_specs=[
            pl.BlockSpec((NP, 2), lambda: (0, 0)),
            pl.BlockSpec((NP, D_OUT), lambda: (0, 0)),
        ],
        out_specs=[
            pl.BlockSpec((NP, D_OUT), lambda: (0, 0)),
            pl.BlockSpec((NP, 1), lambda: (0, 0)),
        ],
        out_shape=[
            jax.ShapeDtypeStruct((NP, D_OUT), jnp.float32),
            jax.ShapeDtypeStruct((NP, 1), jnp.float32),
        ],
    )(degt, h)


def _agg_body(nwps, src_hbm, dst_hbm, g_hbm, accp_hbm,
              acc_sh, sidx_v, didx_v, buf0, buf1, buf2, buf3, zb_v,
              sem_i, sem_j, sem_g0, sem_s0):
    cid = lax.axis_index("c")
    sid = lax.axis_index("s")
    wid = cid * NS + sid

    fs = pltpu.async_copy(src_hbm.at[pl.ds(wid * nwps, nwps), :], sidx_v, sem_i)
    fd = pltpu.async_copy(dst_hbm.at[pl.ds(wid * nwps, nwps), :], didx_v, sem_j)

    @pl.loop(0, WIN)
    def _(i):
        zb_v[i, :] = jnp.zeros((16,), jnp.float32)

    for k in range(ROWS_PER_SUB // WIN):
        pltpu.sync_copy(zb_v, acc_sh.at[pl.ds(sid * ROWS_PER_SUB + k * WIN, WIN)])
    fs.wait()
    fd.wait()
    plsc.subcore_barrier()

    bufs = (buf0, buf1, buf2, buf3)
    for grp in range(0, nwps, 4):
        cnt = min(4, nwps - grp)
        ghs = [
            pltpu.async_copy(g_hbm.at[sidx_v.at[grp + k]], bufs[k], sem_g0)
            for k in range(cnt)
        ]
        for hnd in ghs:
            hnd.wait()
        for k in range(cnt):
            pltpu.async_copy(bufs[k], acc_sh.at[didx_v.at[grp + k]], sem_s0,
                             add=True).wait()

    plsc.subcore_barrier()
    sl = pl.ds(sid * ROWS_PER_SUB, ROWS_PER_SUB)
    pltpu.sync_copy(acc_sh.at[sl], accp_hbm.at[cid, sl])


def _aggregate(src2d, dst2d, g):
    nwps = src2d.shape[0] // NW
    kern = pl.kernel(
        functools.partial(_agg_body, nwps),
        out_type=jax.ShapeDtypeStruct((NC, NP, D_OUT), jnp.float32),
        mesh=_mesh,
        scratch_types=[
            pltpu.VMEM_SHARED((NP, D_OUT), jnp.float32),
            pltpu.VMEM((nwps, WIN), jnp.int32),
            pltpu.VMEM((nwps, WIN), jnp.int32),
            pltpu.VMEM((WIN, D_OUT), jnp.float32),
            pltpu.VMEM((WIN, D_OUT), jnp.float32),
            pltpu.VMEM((WIN, D_OUT), jnp.float32),
            pltpu.VMEM((WIN, D_OUT), jnp.float32),
            pltpu.VMEM((WIN, D_OUT), jnp.float32),
        ] + [pltpu.SemaphoreType.DMA] * 4,
        compiler_params=_sc_params,
    )
    return kern(src2d, dst2d, g)


def _epilogue_body(dinv_ref, g_ref, a0_ref, a1_ref, b_ref, out_ref):
    acc = g_ref[:, :] + a0_ref[:, :] + a1_ref[:, :]
    out_ref[:, :] = dinv_ref[:, :] * acc + b_ref[:, :]


def _epilogue(dinv, g, accp, b2):
    full = lambda: (0, 0)
    return pl.pallas_call(
        _epilogue_body,
        in_specs=[
            pl.BlockSpec((NP, 1), full),
            pl.BlockSpec((NP, D_OUT), full),
            pl.BlockSpec((NP, D_OUT), full),
            pl.BlockSpec((NP, D_OUT), full),
            pl.BlockSpec((1, D_OUT), full),
        ],
        out_specs=pl.BlockSpec((NP, D_OUT), full),
        out_shape=jax.ShapeDtypeStruct((NP, D_OUT), jnp.float32),
    )(dinv, g, accp[0], accp[1], b2)


def kernel(x, edge_index, W, b):
    n, e = x.shape[0], edge_index.shape[1]
    src = edge_index[0].astype(jnp.int32)
    dst = edge_index[1].astype(jnp.int32)
    ep = -(-e // E_TILE) * E_TILE
    if ep != e:
        sent = jnp.full((ep - e,), NP - 1, jnp.int32)
        src = jnp.concatenate([src, sent])
        dst = jnp.concatenate([dst, sent])
    src2d = src.reshape(ep // WIN, WIN)
    dst2d = dst.reshape(ep // WIN, WIN)

    x_p = jnp.pad(x, ((0, NP - n), (0, 0)))
    h = _matmul(x_p, W)
    degp = _degrees(dst2d)
    degt = jnp.transpose(degp)
    g, dinv = _scale(degt, h)
    accp = _aggregate(src2d, dst2d, g)
    out_p = _epilogue(dinv, g, accp, jnp.reshape(b, (1, D_OUT)))
    return out_p[:n]


# speedup vs baseline: 29.1269x; 1.0369x over previous
import functools

import jax
import jax.numpy as jnp
from jax import lax
from jax.experimental import pallas as pl
from jax.experimental.pallas import tpu as pltpu
from jax.experimental.pallas import tpu_sc as plsc

NP = 10240
D_FEAT = 256
D_OUT = 16
WIN = 128
NC = 2
NS = 16
NW = NC * NS
WPS = 40
E_TILE = NW * WPS * WIN
ROWS_PER_SUB = NP // NS

_mesh = plsc.VectorSubcoreMesh(core_axis_name="c", subcore_axis_name="s")
_sc_params = pltpu.CompilerParams(use_tc_tiling_on_sc=False)


_MM_BLK = 1280


def _mm_scale_body(n, x_ref, w_ref, degt_ref, g_ref, dinv_ref):
    i = pl.program_id(0)
    h = lax.dot_general(
        x_ref[:, :], w_ref[:, :], (((1,), (0,)), ((), ())),
        precision=lax.Precision.HIGHEST, preferred_element_type=jnp.float32)
    row = i * _MM_BLK + lax.broadcasted_iota(jnp.int32, (_MM_BLK, 1), 0)
    h = jnp.where(row < n, h, 0.0)
    deg = degt_ref[:, 0:1] + degt_ref[:, 1:2] + 1.0
    dinv = lax.rsqrt(deg)
    dinv_ref[:, :] = dinv
    g_ref[:, :] = h * dinv


def _mm_scale(n, x, W, degt):
    return pl.pallas_call(
        functools.partial(_mm_scale_body, n),
        grid=(NP // _MM_BLK,),
        in_specs=[
            pl.BlockSpec((_MM_BLK, D_FEAT), lambda i: (i, 0)),
            pl.BlockSpec((D_FEAT, D_OUT), lambda i: (0, 0)),
            pl.BlockSpec((_MM_BLK, 2), lambda i: (i, 0)),
        ],
        out_specs=[
            pl.BlockSpec((_MM_BLK, D_OUT), lambda i: (i, 0)),
            pl.BlockSpec((_MM_BLK, 1), lambda i: (i, 0)),
        ],
        out_shape=[
            jax.ShapeDtypeStruct((NP, D_OUT), jnp.float32),
            jax.ShapeDtypeStruct((NP, 1), jnp.float32),
        ],
    )(x, W, degt)


def _deg_body(nwps, dst_hbm, degp_hbm, deg_sh, idx_v, ones_v, zero_v,
              sem_i, sem_w):
    cid = lax.axis_index("c")
    sid = lax.axis_index("s")
    wid = cid * NS + sid

    fetch = pltpu.async_copy(dst_hbm.at[pl.ds(wid * nwps, nwps), :], idx_v,
                             sem_i)

    @pl.loop(0, ROWS_PER_SUB, step=16)
    def _(i):
        zero_v[pl.ds(i, 16)] = jnp.zeros((16,), jnp.float32)

    @pl.loop(0, WIN, step=16)
    def _(i):
        ones_v[pl.ds(i, 16)] = jnp.ones((16,), jnp.float32)

    pltpu.sync_copy(zero_v, deg_sh.at[pl.ds(sid * ROWS_PER_SUB, ROWS_PER_SUB)])
    fetch.wait()
    plsc.subcore_barrier()

    for j in range(nwps):
        pltpu.async_copy(ones_v, deg_sh.at[idx_v.at[j]], sem_w,
                         add=True).wait()

    plsc.subcore_barrier()
    sl = pl.ds(sid * ROWS_PER_SUB, ROWS_PER_SUB)
    pltpu.sync_copy(deg_sh.at[sl], degp_hbm.at[cid, sl])


def _degrees(dst2d):
    nwps = dst2d.shape[0] // NW
    kern = pl.kernel(
        functools.partial(_deg_body, nwps),
        out_type=jax.ShapeDtypeStruct((NC, NP), jnp.float32),
        mesh=_mesh,
        scratch_types=[
            pltpu.VMEM_SHARED((NP,), jnp.float32),
            pltpu.VMEM((nwps, WIN), jnp.int32),
            pltpu.VMEM((WIN,), jnp.float32),
            pltpu.VMEM((ROWS_PER_SUB,), jnp.float32),
            pltpu.SemaphoreType.DMA,
            pltpu.SemaphoreType.DMA,
        ],
        compiler_params=_sc_params,
    )
    return kern(dst2d)


def _agg_body(nwps, src_hbm, dst_hbm, g_hbm, accp_hbm,
              acc_sh, sidx_v, didx_v, buf0, buf1, buf2, buf3, zb_v,
              sem_i, sem_j, sem_g0, sem_s0):
    cid = lax.axis_index("c")
    sid = lax.axis_index("s")
    wid = cid * NS + sid

    fs = pltpu.async_copy(src_hbm.at[pl.ds(wid * nwps, nwps), :], sidx_v, sem_i)
    fd = pltpu.async_copy(dst_hbm.at[pl.ds(wid * nwps, nwps), :], didx_v, sem_j)

    @pl.loop(0, WIN)
    def _(i):
        zb_v[i, :] = jnp.zeros((16,), jnp.float32)

    for k in range(ROWS_PER_SUB // WIN):
        pltpu.sync_copy(zb_v, acc_sh.at[pl.ds(sid * ROWS_PER_SUB + k * WIN, WIN)])
    fs.wait()
    fd.wait()
    plsc.subcore_barrier()

    bufs = (buf0, buf1, buf2, buf3)
    for grp in range(0, nwps, 4):
        cnt = min(4, nwps - grp)
        ghs = [
            pltpu.async_copy(g_hbm.at[sidx_v.at[grp + k]], bufs[k], sem_g0)
            for k in range(cnt)
        ]
        for hnd in ghs:
            hnd.wait()
        for k in range(cnt):
            pltpu.async_copy(bufs[k], acc_sh.at[didx_v.at[grp + k]], sem_s0,
                             add=True).wait()

    plsc.subcore_barrier()
    sl = pl.ds(sid * ROWS_PER_SUB, ROWS_PER_SUB)
    pltpu.sync_copy(acc_sh.at[sl], accp_hbm.at[cid, sl])


def _aggregate(src2d, dst2d, g):
    nwps = src2d.shape[0] // NW
    kern = pl.kernel(
        functools.partial(_agg_body, nwps),
        out_type=jax.ShapeDtypeStruct((NC, NP, D_OUT), jnp.float32),
        mesh=_mesh,
        scratch_types=[
            pltpu.VMEM_SHARED((NP, D_OUT), jnp.float32),
            pltpu.VMEM((nwps, WIN), jnp.int32),
            pltpu.VMEM((nwps, WIN), jnp.int32),
            pltpu.VMEM((WIN, D_OUT), jnp.float32),
            pltpu.VMEM((WIN, D_OUT), jnp.float32),
            pltpu.VMEM((WIN, D_OUT), jnp.float32),
            pltpu.VMEM((WIN, D_OUT), jnp.float32),
            pltpu.VMEM((WIN, D_OUT), jnp.float32),
        ] + [pltpu.SemaphoreType.DMA] * 4,
        compiler_params=_sc_params,
    )
    return kern(src2d, dst2d, g)


def _epilogue_body(n, dinv_ref, g_ref, a0_ref, a1_ref, b_ref, out_ref):
    acc = g_ref[:, :] + a0_ref[:, :] + a1_ref[:, :]
    out = dinv_ref[:, :] * acc + b_ref[:, :]
    out_ref[:, :] = out[:n]


def _epilogue(n, dinv, g, accp, b2):
    full = lambda: (0, 0)
    return pl.pallas_call(
        functools.partial(_epilogue_body, n),
        in_specs=[
            pl.BlockSpec((NP, 1), full),
            pl.BlockSpec((NP, D_OUT), full),
            pl.BlockSpec((NP, D_OUT), full),
            pl.BlockSpec((NP, D_OUT), full),
            pl.BlockSpec((1, D_OUT), full),
        ],
        out_specs=pl.BlockSpec((n, D_OUT), full),
        out_shape=jax.ShapeDtypeStruct((n, D_OUT), jnp.float32),
    )(dinv, g, accp[0], accp[1], b2)


def kernel(x, edge_index, W, b):
    n, e = x.shape[0], edge_index.shape[1]
    src = edge_index[0].astype(jnp.int32)
    dst = edge_index[1].astype(jnp.int32)
    ep = -(-e // E_TILE) * E_TILE
    if ep != e:
        sent = jnp.full((ep - e,), NP - 1, jnp.int32)
        src = jnp.concatenate([src, sent])
        dst = jnp.concatenate([dst, sent])
    src2d = src.reshape(ep // WIN, WIN)
    dst2d = dst.reshape(ep // WIN, WIN)

    degp = _degrees(dst2d)
    degt = jnp.transpose(degp)
    g, dinv = _mm_scale(n, x, W, degt)
    accp = _aggregate(src2d, dst2d, g)
    return _epilogue(n, dinv, g, accp, jnp.reshape(b, (1, D_OUT)))


# speedup vs baseline: 29.6383x; 1.0176x over previous
import functools

import jax
import jax.numpy as jnp
from jax import lax
from jax.experimental import pallas as pl
from jax.experimental.pallas import tpu as pltpu
from jax.experimental.pallas import tpu_sc as plsc

NP = 10240
D_FEAT = 256
D_OUT = 16
WIN = 128
NC = 2
NS = 16
NW = NC * NS
WPS = 40
E_TILE = NW * WPS * WIN
ROWS_PER_SUB = NP // NS

_mesh = plsc.VectorSubcoreMesh(core_axis_name="c", subcore_axis_name="s")
_sc_params = pltpu.CompilerParams(use_tc_tiling_on_sc=False)


_MM_BLK = 1280


def _mm_scale_body(n, x_ref, w_ref, degt_ref, g_ref, dinv_ref):
    i = pl.program_id(0)
    h = lax.dot_general(
        x_ref[:, :], w_ref[:, :], (((1,), (0,)), ((), ())),
        precision=lax.Precision.HIGHEST, preferred_element_type=jnp.float32)
    row = i * _MM_BLK + lax.broadcasted_iota(jnp.int32, (_MM_BLK, 1), 0)
    h = jnp.where(row < n, h, 0.0)
    deg = degt_ref[:, 0:1] + degt_ref[:, 1:2] + 1.0
    dinv = lax.rsqrt(deg)
    dinv_ref[:, :] = dinv
    g_ref[:, :] = h * dinv


def _mm_scale(n, x, W, degt):
    return pl.pallas_call(
        functools.partial(_mm_scale_body, n),
        grid=(NP // _MM_BLK,),
        in_specs=[
            pl.BlockSpec((_MM_BLK, D_FEAT), lambda i: (i, 0)),
            pl.BlockSpec((D_FEAT, D_OUT), lambda i: (0, 0)),
            pl.BlockSpec((_MM_BLK, 2), lambda i: (i, 0)),
        ],
        out_specs=[
            pl.BlockSpec((_MM_BLK, D_OUT), lambda i: (i, 0)),
            pl.BlockSpec((_MM_BLK, 1), lambda i: (i, 0)),
        ],
        out_shape=[
            jax.ShapeDtypeStruct((NP, D_OUT), jnp.float32),
            jax.ShapeDtypeStruct((NP, 1), jnp.float32),
        ],
    )(x, W, degt)


def _deg_body(nwps, dst_hbm, degp_hbm, deg_sh, idx_v, ones_v, zero_v,
              sem_i, sem_w):
    cid = lax.axis_index("c")
    sid = lax.axis_index("s")
    wid = cid * NS + sid

    fetch = pltpu.async_copy(dst_hbm.at[pl.ds(wid * nwps, nwps), :], idx_v,
                             sem_i)

    @pl.loop(0, ROWS_PER_SUB, step=16)
    def _(i):
        zero_v[pl.ds(i, 16)] = jnp.zeros((16,), jnp.float32)

    @pl.loop(0, WIN, step=16)
    def _(i):
        ones_v[pl.ds(i, 16)] = jnp.ones((16,), jnp.float32)

    pltpu.sync_copy(zero_v, deg_sh.at[pl.ds(sid * ROWS_PER_SUB, ROWS_PER_SUB)])
    fetch.wait()
    plsc.subcore_barrier()

    for j in range(nwps):
        pltpu.async_copy(ones_v, deg_sh.at[idx_v.at[j]], sem_w,
                         add=True).wait()

    plsc.subcore_barrier()
    sl = pl.ds(sid * ROWS_PER_SUB, ROWS_PER_SUB)
    pltpu.sync_copy(deg_sh.at[sl], degp_hbm.at[cid, sl])


def _degrees(dst2d):
    nwps = dst2d.shape[0] // NW
    kern = pl.kernel(
        functools.partial(_deg_body, nwps),
        out_type=jax.ShapeDtypeStruct((NC, NP), jnp.float32),
        mesh=_mesh,
        scratch_types=[
            pltpu.VMEM_SHARED((NP,), jnp.float32),
            pltpu.VMEM((nwps, WIN), jnp.int32),
            pltpu.VMEM((WIN,), jnp.float32),
            pltpu.VMEM((ROWS_PER_SUB,), jnp.float32),
            pltpu.SemaphoreType.DMA,
            pltpu.SemaphoreType.DMA,
        ],
        compiler_params=_sc_params,
    )
    return kern(dst2d)


def _agg_body(nwps, src_hbm, dst_hbm, g_hbm, accp_hbm,
              acc_sh, sidx_v, didx_v, rbuf, zb_v,
              sem_i, sem_j, sem_g0, sem_s0):
    cid = lax.axis_index("c")
    sid = lax.axis_index("s")
    wid = cid * NS + sid

    fs = pltpu.async_copy(src_hbm.at[pl.ds(wid * nwps, nwps), :], sidx_v, sem_i)
    fd = pltpu.async_copy(dst_hbm.at[pl.ds(wid * nwps, nwps), :], didx_v, sem_j)

    @pl.loop(0, WIN)
    def _(i):
        zb_v[i, :] = jnp.zeros((16,), jnp.float32)

    for k in range(ROWS_PER_SUB // WIN):
        pltpu.sync_copy(zb_v, acc_sh.at[pl.ds(sid * ROWS_PER_SUB + k * WIN, WIN)])
    fs.wait()
    fd.wait()
    plsc.subcore_barrier()

    for grp in range(0, nwps, 8):
        cnt = min(8, nwps - grp)
        ghs = [
            pltpu.async_copy(g_hbm.at[sidx_v.at[grp + k]], rbuf.at[k], sem_g0)
            for k in range(cnt)
        ]
        for hnd in ghs:
            hnd.wait()
        for sg in range(0, cnt, 4):
            shs = [
                pltpu.async_copy(rbuf.at[sg + k], acc_sh.at[didx_v.at[grp + sg + k]],
                                 sem_s0, add=True)
                for k in range(min(4, cnt - sg))
            ]
            for hnd in shs:
                hnd.wait()

    plsc.subcore_barrier()
    sl = pl.ds(sid * ROWS_PER_SUB, ROWS_PER_SUB)
    pltpu.sync_copy(acc_sh.at[sl], accp_hbm.at[cid, sl])


def _aggregate(src2d, dst2d, g):
    nwps = src2d.shape[0] // NW
    kern = pl.kernel(
        functools.partial(_agg_body, nwps),
        out_type=jax.ShapeDtypeStruct((NC, NP, D_OUT), jnp.float32),
        mesh=_mesh,
        scratch_types=[
            pltpu.VMEM_SHARED((NP, D_OUT), jnp.float32),
            pltpu.VMEM((nwps, WIN), jnp.int32),
            pltpu.VMEM((nwps, WIN), jnp.int32),
            pltpu.VMEM((8, WIN, D_OUT), jnp.float32),
            pltpu.VMEM((WIN, D_OUT), jnp.float32),
        ] + [pltpu.SemaphoreType.DMA] * 4,
        compiler_params=_sc_params,
    )
    return kern(src2d, dst2d, g)


def _epilogue_body(n, dinv_ref, g_ref, a0_ref, a1_ref, b_ref, out_ref):
    acc = g_ref[:, :] + a0_ref[:, :] + a1_ref[:, :]
    out = dinv_ref[:, :] * acc + b_ref[:, :]
    out_ref[:, :] = out[:n]


def _epilogue(n, dinv, g, accp, b2):
    full = lambda: (0, 0)
    return pl.pallas_call(
        functools.partial(_epilogue_body, n),
        in_specs=[
            pl.BlockSpec((NP, 1), full),
            pl.BlockSpec((NP, D_OUT), full),
            pl.BlockSpec((NP, D_OUT), full),
            pl.BlockSpec((NP, D_OUT), full),
            pl.BlockSpec((1, D_OUT), full),
        ],
        out_specs=pl.BlockSpec((n, D_OUT), full),
        out_shape=jax.ShapeDtypeStruct((n, D_OUT), jnp.float32),
    )(dinv, g, accp[0], accp[1], b2)


def kernel(x, edge_index, W, b):
    n, e = x.shape[0], edge_index.shape[1]
    src = edge_index[0].astype(jnp.int32)
    dst = edge_index[1].astype(jnp.int32)
    ep = -(-e // E_TILE) * E_TILE
    if ep != e:
        sent = jnp.full((ep - e,), NP - 1, jnp.int32)
        src = jnp.concatenate([src, sent])
        dst = jnp.concatenate([dst, sent])
    src2d = src.reshape(ep // WIN, WIN)
    dst2d = dst.reshape(ep // WIN, WIN)

    degp = _degrees(dst2d)
    degt = jnp.transpose(degp)
    g, dinv = _mm_scale(n, x, W, degt)
    accp = _aggregate(src2d, dst2d, g)
    return _epilogue(n, dinv, g, accp, jnp.reshape(b, (1, D_OUT)))
